# baseline (speedup 1.0000x reference)
import functools

import jax
import jax.numpy as jnp
from jax import lax
from jax.experimental import pallas as pl
from jax.experimental.pallas import tpu as pltpu
from jax.experimental.pallas import tpu_sc as plsc

H = 64
W = 64
D = 256
HALF = D // 2
LANES = 16

_GATHER_1D = lax.GatherDimensionNumbers(
    offset_dims=(), collapsed_slice_dims=(0,), start_index_map=(0,))


def _splat_lane(v16, lane):
    idx = jnp.full((LANES,), lane, jnp.int32)
    return lax.gather(v16, idx[:, None], _GATHER_1D, slice_sizes=(1,),
                      mode=lax.GatherScatterMode.PROMISE_IN_BOUNDS)


def _build_pos_kernel(batch):
    info = plsc.get_sparse_core_info()
    nc, ns = info.num_cores, info.num_subcores
    nw = nc * ns
    ch_per_w = D // nw
    n_col_workers = HALF // ch_per_w
    plane_w = H * W
    mesh = plsc.VectorSubcoreMesh(core_axis_name="c", subcore_axis_name="s")

    @functools.partial(
        pl.kernel,
        mesh=mesh,
        out_type=jax.ShapeDtypeStruct((batch * D * plane_w,), jnp.float32),
        scratch_types=[
            pltpu.VMEM((HALF * H,), jnp.float32),
            pltpu.VMEM((ch_per_w * plane_w,), jnp.float32),
            pltpu.SemaphoreType.DMA,
        ],
    )
    def pos_kernel(row_t_hbm, col_t_hbm, out_hbm, table_v, plane_v, sem):
        cid = lax.axis_index("c")
        sid = lax.axis_index("s")
        wid = sid * nc + cid
        is_col = wid < n_col_workers
        base = jnp.where(is_col, wid * ch_per_w, wid * ch_per_w - HALF)

        @pl.when(is_col)
        def _():
            pltpu.sync_copy(col_t_hbm, table_v)

        @pl.when(jnp.logical_not(is_col))
        def _():
            pltpu.sync_copy(row_t_hbm, table_v)

        @pl.when(is_col)
        def _():
            for j in range(ch_per_w):
                chunks = [
                    table_v[pl.ds((base + j) * H + LANES * xc, LANES)]
                    for xc in range(W // LANES)
                ]

                def body(y, carry, j=j, chunks=chunks):
                    off = j * plane_w + y * W
                    for xc in range(W // LANES):
                        plane_v[pl.ds(off + LANES * xc, LANES)] = chunks[xc]
                    return carry

                lax.fori_loop(0, H, body, 0)

        @pl.when(jnp.logical_not(is_col))
        def _():
            for j in range(ch_per_w):
                def body(yc, carry, j=j):
                    v16 = table_v[pl.ds((base + j) * H + yc * LANES, LANES)]
                    for lane in range(LANES):
                        vec = _splat_lane(v16, lane)
                        off = j * plane_w + (yc * LANES + lane) * W
                        for xc in range(W // LANES):
                            plane_v[pl.ds(off + LANES * xc, LANES)] = vec
                    return carry

                lax.fori_loop(0, H // LANES, body, 0)

        chan = wid * ch_per_w
        copies = [
            pltpu.async_copy(
                plane_v,
                out_hbm.at[pl.ds((b * D + chan) * plane_w, ch_per_w * plane_w)],
                sem)
            for b in range(batch)
        ]
        for cp in copies:
            cp.wait()

    return pos_kernel


def kernel(x, row_embed, col_embed):
    row_t = row_embed.T.reshape(-1)
    col_t = col_embed.T.reshape(-1)
    batch = x.shape[0]
    flat = _build_pos_kernel(batch)(row_t, col_t)
    return flat.reshape(batch, D, H, W)
